# Initial kernel scaffold; baseline (speedup 1.0000x reference)
import jax
import jax.numpy as jnp
from jax.experimental import pallas as pl


def kernel(pitch, amplitude, W, attention, sec):
    raise NotImplementedError("write your pallas kernel here")


# speedup vs baseline: 15.7895x; 15.7895x over previous
import functools

import jax
import jax.numpy as jnp
from jax import lax
from jax.experimental import pallas as pl
from jax.experimental.pallas import tpu as pltpu
from jax.experimental.pallas import tpu_sc as plsc

_SR = 16000.0
_WTLEN = 512
_BLOCK = 160
_NFIXED = 4
_NTILES = 32
_LANES = 16


def _tc_prep_body(pitch_ref, w_ref, att_ref, m_ref, car_ref, *, nchunk):
    w = w_ref[...]
    rid = lax.broadcasted_iota(jnp.int32, w.shape, 0)
    proc = jnp.where(rid < _NFIXED, w, jnp.tanh(w))
    a = att_ref[...]
    a = a - jnp.max(a, axis=0, keepdims=True)
    e = jnp.exp(a)
    att = e / jnp.sum(e, axis=0, keepdims=True)
    m_ref[...] = lax.dot_general(
        att, proc, (((0,), (0,)), ((), ())),
        preferred_element_type=jnp.float32,
        precision=lax.Precision.HIGHEST,
    )
    inc = pitch_ref[...] / _SR * _WTLEN
    sums = jnp.sum(inc, axis=1, keepdims=True)
    i_ = lax.broadcasted_iota(jnp.int32, (nchunk, nchunk), 0)
    j_ = lax.broadcasted_iota(jnp.int32, (nchunk, nchunk), 1)
    ltri = (j_ < i_).astype(jnp.float32)
    pref = lax.dot_general(
        ltri, sums, (((1,), (0,)), ((), ())),
        preferred_element_type=jnp.float32,
        precision=lax.Precision.HIGHEST,
    )
    car_ref[...] = jnp.broadcast_to(pref, (nchunk, _LANES))


def _sc_synth_body(pitch_hbm, amp_hbm, m_hbm, car_hbm, out_hbm,
                   pitch_v, amp_v, out_v, m_v, car_v,
                   *, chunk, nrows, nblocks, wtlen):
    wid = lax.axis_index("s") * 2 + lax.axis_index("c")
    t0 = wid * chunk
    pltpu.sync_copy(pitch_hbm.at[pl.ds(t0, chunk)], pitch_v)
    pltpu.sync_copy(amp_hbm.at[pl.ds(t0, chunk)], amp_v)
    pltpu.sync_copy(car_hbm.at[pl.ds(wid * _LANES, _LANES)], car_v)
    row0 = jnp.minimum(t0 // _BLOCK, nblocks - nrows)
    pltpu.sync_copy(m_hbm.at[pl.ds(row0 * wtlen, nrows * wtlen)], m_v)

    lane = lax.iota(jnp.int32, _LANES)
    zeros = jnp.zeros((_LANES,), jnp.float32)
    vec_wtlen_f = jnp.full((_LANES,), float(_WTLEN), jnp.float32)
    vec_mask_i = jnp.full((_LANES,), _WTLEN - 1, jnp.int32)
    vec_one_i = jnp.full((_LANES,), 1, jnp.int32)
    sr_v = jnp.full((_LANES,), _SR, jnp.float32)
    wt_f = jnp.full((_LANES,), float(_WTLEN), jnp.float32)
    last_idx = jnp.full((_LANES,), _LANES - 1, jnp.int32)

    def _take(v, idx):
        return lax.gather(
            v, idx[:, None],
            dimension_numbers=lax.GatherDimensionNumbers(
                offset_dims=(), collapsed_slice_dims=(0,),
                start_index_map=(0,)),
            slice_sizes=(1,),
            mode=lax.GatherScatterMode.PROMISE_IN_BOUNDS)

    def _lane_cumsum(x):
        y = x
        for k in (1, 2, 4, 8):
            g = _take(y, jnp.maximum(lane - k, 0))
            y = y + jnp.where(lane >= k, g, zeros)
        return y

    carry0 = car_v[...]

    def body(i, carry):
        p = pitch_v[pl.ds(i * _LANES, _LANES)]
        inc = p / sr_v * wt_f
        tot = _lane_cumsum(inc) + carry
        idx = tot - inc
        idxm = lax.rem(idx, vec_wtlen_f)
        il = idxm.astype(jnp.int32)
        alpha = idxm - il.astype(jnp.float32)
        ih = (il + vec_one_i) & vec_mask_i
        row = jnp.minimum(lax.div(t0 + i * _LANES, _BLOCK), nblocks - 1) - row0
        base = jnp.full((_LANES,), row * wtlen, dtype=jnp.int32)
        lo = plsc.load_gather(m_v, [base + il])
        hi = plsc.load_gather(m_v, [base + ih])
        amp = amp_v[pl.ds(i * _LANES, _LANES)]
        out_v[pl.ds(i * _LANES, _LANES)] = (lo + alpha * (hi - lo)) * amp
        return _take(tot, last_idx)

    lax.fori_loop(0, chunk // _LANES, body, carry0)
    pltpu.sync_copy(out_v, out_hbm.at[pl.ds(t0, chunk)])


@jax.jit
def kernel(pitch, amplitude, W, attention, sec):
    t = pitch.shape[1]
    nwt, wtlen = W.shape
    nblocks = attention.shape[1]
    chunk = -(-t // (_NTILES * _LANES)) * _LANES
    tpad = _NTILES * chunk
    nrows = chunk // _BLOCK + 3

    pitch_p = jnp.zeros((tpad,), jnp.float32).at[:t].set(pitch[0])
    amp_p = jnp.zeros((tpad,), jnp.float32).at[:t].set(amplitude[:, 0])

    m, carries = pl.pallas_call(
        functools.partial(_tc_prep_body, nchunk=_NTILES),
        out_shape=(
            jax.ShapeDtypeStruct((nblocks, wtlen), jnp.float32),
            jax.ShapeDtypeStruct((_NTILES, _LANES), jnp.float32),
        ),
    )(pitch_p.reshape(_NTILES, chunk), W, attention)

    mesh = plsc.VectorSubcoreMesh(core_axis_name="c", subcore_axis_name="s")
    sc = functools.partial(
        pl.kernel,
        mesh=mesh,
        compiler_params=pltpu.CompilerParams(needs_layout_passes=False),
        out_type=jax.ShapeDtypeStruct((tpad,), jnp.float32),
        scratch_types=[
            pltpu.VMEM((chunk,), jnp.float32),
            pltpu.VMEM((chunk,), jnp.float32),
            pltpu.VMEM((chunk,), jnp.float32),
            pltpu.VMEM((nrows * wtlen,), jnp.float32),
            pltpu.VMEM((_LANES,), jnp.float32),
        ],
    )(functools.partial(_sc_synth_body, chunk=chunk, nrows=nrows,
                        nblocks=nblocks, wtlen=wtlen))
    out = sc(pitch_p, amp_p, m.reshape(nblocks * wtlen), carries.reshape(-1))

    return out[:t].reshape(1, t, 1)


# speedup vs baseline: 16.1768x; 1.0245x over previous
import functools

import jax
import jax.numpy as jnp
from jax import lax
from jax.experimental import pallas as pl
from jax.experimental.pallas import tpu as pltpu
from jax.experimental.pallas import tpu_sc as plsc

_SR = 16000.0
_WTLEN = 512
_BLOCK = 160
_NFIXED = 4
_NTILES = 32
_LANES = 16


def _tc_prep_body(pitch_ref, w_ref, att_ref, m_ref, car_ref, *, nchunk):
    w = w_ref[...]
    rid = lax.broadcasted_iota(jnp.int32, w.shape, 0)
    proc = jnp.where(rid < _NFIXED, w, jnp.tanh(w))
    a = att_ref[...]
    a = a - jnp.max(a, axis=0, keepdims=True)
    e = jnp.exp(a)
    att = e / jnp.sum(e, axis=0, keepdims=True)
    m_ref[...] = lax.dot_general(
        att, proc, (((0,), (0,)), ((), ())),
        preferred_element_type=jnp.float32,
        precision=lax.Precision.HIGHEST,
    )
    inc = pitch_ref[...] / _SR * _WTLEN
    sums = jnp.sum(inc, axis=1, keepdims=True)
    i_ = lax.broadcasted_iota(jnp.int32, (nchunk, nchunk), 0)
    j_ = lax.broadcasted_iota(jnp.int32, (nchunk, nchunk), 1)
    ltri = (j_ < i_).astype(jnp.float32)
    pref = lax.dot_general(
        ltri, sums, (((1,), (0,)), ((), ())),
        preferred_element_type=jnp.float32,
        precision=lax.Precision.HIGHEST,
    )
    car_ref[...] = jnp.broadcast_to(pref, (nchunk, _LANES))


def _sc_synth_body(pitch_hbm, amp_hbm, m_hbm, car_hbm, out_hbm,
                   pitch_v, amp_v, out_v, m_v, car_v, cs_v, vcar_v,
                   sem_p, sem_a, sem_c, sem_m,
                   *, chunk, nrows, nblocks, wtlen):
    wid = lax.axis_index("s") * 2 + lax.axis_index("c")
    t0 = wid * chunk
    cp_p = pltpu.async_copy(pitch_hbm.at[pl.ds(t0, chunk)], pitch_v, sem_p)
    cp_a = pltpu.async_copy(amp_hbm.at[pl.ds(t0, chunk)], amp_v, sem_a)
    cp_c = pltpu.async_copy(car_hbm.at[pl.ds(wid * _LANES, _LANES)], car_v,
                            sem_c)
    row0 = jnp.minimum(t0 // _BLOCK, nblocks - nrows)
    cp_m = pltpu.async_copy(m_hbm.at[pl.ds(row0 * wtlen, nrows * wtlen)], m_v,
                            sem_m)

    lane = lax.iota(jnp.int32, _LANES)
    zeros = jnp.zeros((_LANES,), jnp.float32)
    vec_wtlen_f = jnp.full((_LANES,), float(_WTLEN), jnp.float32)
    vec_mask_i = jnp.full((_LANES,), _WTLEN - 1, jnp.int32)
    vec_one_i = jnp.full((_LANES,), 1, jnp.int32)
    sr_v = jnp.full((_LANES,), _SR, jnp.float32)
    wt_f = jnp.full((_LANES,), float(_WTLEN), jnp.float32)
    last_idx = jnp.full((_LANES,), _LANES - 1, jnp.int32)
    nvec = chunk // _LANES

    def _take(v, idx):
        return lax.gather(
            v, idx[:, None],
            dimension_numbers=lax.GatherDimensionNumbers(
                offset_dims=(), collapsed_slice_dims=(0,),
                start_index_map=(0,)),
            slice_sizes=(1,),
            mode=lax.GatherScatterMode.PROMISE_IN_BOUNDS)

    def _lane_cumsum(x):
        y = x
        for k in (1, 2, 4, 8):
            g = _take(y, jnp.maximum(lane - k, 0))
            y = y + jnp.where(lane >= k, g, zeros)
        return y

    cp_p.wait()

    @plsc.parallel_loop(0, nvec, unroll=8)
    def _phase_a(i):
        p = pitch_v[pl.ds(i * _LANES, _LANES)]
        inc = p / sr_v * wt_f
        cs_v[pl.ds(i * _LANES, _LANES)] = _lane_cumsum(inc)

    cp_c.wait()

    def _phase_b(i, carry):
        cs = cs_v[pl.ds(i * _LANES, _LANES)]
        vcar_v[pl.ds(i * _LANES, _LANES)] = carry
        return _take(cs, last_idx) + carry

    lax.fori_loop(0, nvec, _phase_b, car_v[...])

    cp_a.wait()
    cp_m.wait()

    @plsc.parallel_loop(0, nvec, unroll=4)
    def _phase_c(i):
        p = pitch_v[pl.ds(i * _LANES, _LANES)]
        inc = p / sr_v * wt_f
        cs = cs_v[pl.ds(i * _LANES, _LANES)]
        carry = vcar_v[pl.ds(i * _LANES, _LANES)]
        idx = cs + carry - inc
        idxm = lax.rem(idx, vec_wtlen_f)
        il = idxm.astype(jnp.int32)
        alpha = idxm - il.astype(jnp.float32)
        ih = (il + vec_one_i) & vec_mask_i
        row = jnp.minimum(lax.div(t0 + i * _LANES, _BLOCK), nblocks - 1) - row0
        base = jnp.full((_LANES,), row * wtlen, dtype=jnp.int32)
        lo = plsc.load_gather(m_v, [base + il])
        hi = plsc.load_gather(m_v, [base + ih])
        amp = amp_v[pl.ds(i * _LANES, _LANES)]
        out_v[pl.ds(i * _LANES, _LANES)] = (lo + alpha * (hi - lo)) * amp

    pltpu.sync_copy(out_v, out_hbm.at[pl.ds(t0, chunk)])


@jax.jit
def kernel(pitch, amplitude, W, attention, sec):
    t = pitch.shape[1]
    nwt, wtlen = W.shape
    nblocks = attention.shape[1]
    chunk = -(-t // (_NTILES * _LANES)) * _LANES
    tpad = _NTILES * chunk
    nrows = chunk // _BLOCK + 3

    pitch_p = jnp.zeros((tpad,), jnp.float32).at[:t].set(pitch[0])
    amp_p = jnp.zeros((tpad,), jnp.float32).at[:t].set(amplitude[:, 0])

    m, carries = pl.pallas_call(
        functools.partial(_tc_prep_body, nchunk=_NTILES),
        out_shape=(
            jax.ShapeDtypeStruct((nblocks, wtlen), jnp.float32),
            jax.ShapeDtypeStruct((_NTILES, _LANES), jnp.float32),
        ),
    )(pitch_p.reshape(_NTILES, chunk), W, attention)

    mesh = plsc.VectorSubcoreMesh(core_axis_name="c", subcore_axis_name="s")
    sc = functools.partial(
        pl.kernel,
        mesh=mesh,
        compiler_params=pltpu.CompilerParams(needs_layout_passes=False),
        out_type=jax.ShapeDtypeStruct((tpad,), jnp.float32),
        scratch_types=[
            pltpu.VMEM((chunk,), jnp.float32),
            pltpu.VMEM((chunk,), jnp.float32),
            pltpu.VMEM((chunk,), jnp.float32),
            pltpu.VMEM((nrows * wtlen,), jnp.float32),
            pltpu.VMEM((_LANES,), jnp.float32),
            pltpu.VMEM((chunk,), jnp.float32),
            pltpu.VMEM((chunk,), jnp.float32),
            pltpu.SemaphoreType.DMA,
            pltpu.SemaphoreType.DMA,
            pltpu.SemaphoreType.DMA,
            pltpu.SemaphoreType.DMA,
        ],
    )(functools.partial(_sc_synth_body, chunk=chunk, nrows=nrows,
                        nblocks=nblocks, wtlen=wtlen))
    out = sc(pitch_p, amp_p, m.reshape(nblocks * wtlen), carries.reshape(-1))

    return out[:t].reshape(1, t, 1)
